# Initial kernel scaffold; baseline (speedup 1.0000x reference)
import jax
import jax.numpy as jnp
from jax.experimental import pallas as pl


def kernel(input, target, class_num):
    raise NotImplementedError("write your pallas kernel here")


# speedup vs baseline: 19.8656x; 19.8656x over previous
import functools

import jax
import jax.numpy as jnp
from jax import lax
from jax.experimental import pallas as pl
from jax.experimental.pallas import tpu as pltpu
from jax.experimental.pallas import tpu_sc as plsc

_IGNORE = 255
_LANES = 16
_CPAD = 32
_CHUNK = 2048


def _iou_counts(in1, tgt1, B, C, HW):
    NC, NS = 2, 16
    NW = NC * NS
    WPB = NW // B
    PPW = HW // WPB
    nchunks = PPW // _CHUNK
    nvec = _CHUNK // _LANES
    counts_len = 3 * _LANES * _CPAD

    mesh = plsc.VectorSubcoreMesh(core_axis_name="c", subcore_axis_name="s")

    @functools.partial(
        pl.kernel,
        out_type=jax.ShapeDtypeStruct((NW * counts_len,), jnp.float32),
        mesh=mesh,
        scratch_types=[
            pltpu.VMEM((C * _CHUNK,), jnp.float32),
            pltpu.VMEM((_CHUNK,), jnp.int32),
            pltpu.VMEM((counts_len,), jnp.float32),
            pltpu.SemaphoreType.DMA,
        ],
        compiler_params=pltpu.CompilerParams(needs_layout_passes=False),
    )
    def _k(in_hbm, tgt_hbm, out_hbm, buf, tbuf, counts, sem):
        cid = lax.axis_index("c")
        sid = lax.axis_index("s")
        wid = sid * NC + cid
        b = wid // WPB
        base = (wid % WPB) * PPW

        zero = jnp.zeros((_LANES,), jnp.float32)

        def zbody(i, _):
            counts[pl.ds(pl.multiple_of(i * _LANES, _LANES), _LANES)] = zero
            return 0

        lax.fori_loop(0, counts_len // _LANES, zbody, 0)

        lane_off = lax.iota(jnp.int32, _LANES) * _CPAD
        ones = jnp.ones((_LANES,), jnp.float32)
        zf = jnp.zeros((_LANES,), jnp.float32)
        zi = jnp.zeros((_LANES,), jnp.int32)

        def chunk_body(g, _):
            p0 = base + g * _CHUNK
            src0 = b * (C * HW) + p0
            descs = [
                pltpu.async_copy(in_hbm.at[pl.ds(src0 + c * HW, _CHUNK)],
                                 buf.at[pl.ds(c * _CHUNK, _CHUNK)], sem)
                for c in range(C)
            ]
            descs.append(
                pltpu.async_copy(tgt_hbm.at[pl.ds(b * HW + p0, _CHUNK)],
                                 tbuf, sem))
            for d in descs:
                d.wait()

            def vec_body(i, _):
                off = pl.multiple_of(i * _LANES, _LANES)
                m = buf[pl.ds(off, _LANES)]
                a = zi
                for c in range(1, C):
                    v = buf[pl.ds(off + c * _CHUNK, _LANES)]
                    gt = v > m
                    m = jnp.maximum(v, m)
                    a = jnp.where(gt, c, a)
                t = tbuf[pl.ds(off, _LANES)]
                valid = t != _IGNORE
                maskf = jnp.where(valid, ones, zf)
                corrf = jnp.where(valid & (a == t), ones, zf)
                ip = lane_off + a
                it = lane_off + jnp.where(valid, t, zi)
                plsc.addupdate_scatter(counts, [ip], corrf)
                plsc.addupdate_scatter(counts, [ip + (_LANES * _CPAD)], maskf)
                plsc.addupdate_scatter(counts, [it + (2 * _LANES * _CPAD)],
                                       maskf)
                return 0

            lax.fori_loop(0, nvec, vec_body, 0)
            return 0

        lax.fori_loop(0, nchunks, chunk_body, 0)
        pltpu.sync_copy(counts,
                        out_hbm.at[pl.ds(wid * counts_len, counts_len)])

    return _k(in1, tgt1)


def kernel(input, target, class_num):
    B, C, H, W = input.shape
    HW = H * W
    in1 = input.reshape(-1)
    tgt1 = target.reshape(-1)
    partials = _iou_counts(in1, tgt1, B, C, HW)
    p = partials.reshape(-1, 3, _LANES, _CPAD).sum(axis=(0, 2))
    intersect = p[0, :C]
    union = p[1, :C] + p[2, :C] - intersect
    eps = 1e-4
    return (intersect + eps) / (union + eps)


# speedup vs baseline: 25.2013x; 1.2686x over previous
import functools

import jax
import jax.numpy as jnp
from jax import lax
from jax.experimental import pallas as pl
from jax.experimental.pallas import tpu as pltpu
from jax.experimental.pallas import tpu_sc as plsc

_IGNORE = 255
_LANES = 16
_CPAD = 32
_CHUNK = 2048


def _iou_counts(in1, tgt1, B, C, HW):
    NC, NS = 2, 16
    NW = NC * NS
    WPB = NW // B
    PPW = HW // WPB
    nchunks = PPW // _CHUNK
    nvec = _CHUNK // _LANES
    counts_len = 3 * _LANES * _CPAD

    mesh = plsc.VectorSubcoreMesh(core_axis_name="c", subcore_axis_name="s")

    @functools.partial(
        pl.kernel,
        out_type=jax.ShapeDtypeStruct((NW * counts_len,), jnp.float32),
        mesh=mesh,
        scratch_types=[
            pltpu.VMEM((2 * C * _CHUNK,), jnp.float32),
            pltpu.VMEM((2 * _CHUNK,), jnp.int32),
            pltpu.VMEM((counts_len,), jnp.float32),
            pltpu.SemaphoreType.DMA,
            pltpu.SemaphoreType.DMA,
        ],
        compiler_params=pltpu.CompilerParams(needs_layout_passes=False),
    )
    def _k(in_hbm, tgt_hbm, out_hbm, buf, tbuf, counts, sem0, sem1):
        sems = (sem0, sem1)
        cid = lax.axis_index("c")
        sid = lax.axis_index("s")
        wid = sid * NC + cid
        b = wid // WPB
        base = (wid % WPB) * PPW

        zero = jnp.zeros((_LANES,), jnp.float32)

        def zbody(i, _):
            counts[pl.ds(pl.multiple_of(i * _LANES, _LANES), _LANES)] = zero
            return 0

        lax.fori_loop(0, counts_len // _LANES, zbody, 0)

        lane_off = lax.iota(jnp.int32, _LANES) * _CPAD
        ones = jnp.ones((_LANES,), jnp.float32)
        zf = jnp.zeros((_LANES,), jnp.float32)
        zi = jnp.zeros((_LANES,), jnp.int32)

        def fire(g, slot):
            p0 = base + g * _CHUNK
            src0 = b * (C * HW) + p0
            for c in range(C):
                pltpu.async_copy(
                    in_hbm.at[pl.ds(src0 + c * HW, _CHUNK)],
                    buf.at[pl.ds(slot * C * _CHUNK + c * _CHUNK, _CHUNK)],
                    sems[slot])
            pltpu.async_copy(tgt_hbm.at[pl.ds(b * HW + p0, _CHUNK)],
                             tbuf.at[pl.ds(slot * _CHUNK, _CHUNK)],
                             sems[slot])

        def drain(slot):
            pltpu.make_async_copy(
                in_hbm.at[pl.ds(0, C * _CHUNK)],
                buf.at[pl.ds(slot * C * _CHUNK, C * _CHUNK)],
                sems[slot]).wait()
            pltpu.make_async_copy(
                tgt_hbm.at[pl.ds(0, _CHUNK)],
                tbuf.at[pl.ds(slot * _CHUNK, _CHUNK)],
                sems[slot]).wait()

        def compute(slot):
            boff = slot * C * _CHUNK

            def vec_body(i, _):
                off = pl.multiple_of(i * _LANES, _LANES)
                m = buf[pl.ds(off + boff, _LANES)]
                a = zi
                for c in range(1, C):
                    v = buf[pl.ds(off + boff + c * _CHUNK, _LANES)]
                    gt = v > m
                    m = jnp.maximum(v, m)
                    a = jnp.where(gt, c, a)
                t = tbuf[pl.ds(off + slot * _CHUNK, _LANES)]
                valid = t != _IGNORE
                maskf = jnp.where(valid, ones, zf)
                corrf = jnp.where(valid & (a == t), ones, zf)
                ip = lane_off + a
                it = lane_off + jnp.where(valid, t, zi)
                plsc.addupdate_scatter(counts, [ip], corrf)
                plsc.addupdate_scatter(counts, [ip + (_LANES * _CPAD)], maskf)
                plsc.addupdate_scatter(counts, [it + (2 * _LANES * _CPAD)],
                                       maskf)
                return 0

            lax.fori_loop(0, nvec, vec_body, 0)

        fire(0, 0)
        fire(1, 1)

        def pair_body(i, _):
            g0 = 2 * i
            for slot in range(2):
                drain(slot)
                compute(slot)

                @pl.when(g0 + slot + 2 < nchunks)
                def _fire_next(slot=slot):
                    fire(g0 + slot + 2, slot)
            return 0

        lax.fori_loop(0, nchunks // 2, pair_body, 0)
        pltpu.sync_copy(counts,
                        out_hbm.at[pl.ds(wid * counts_len, counts_len)])

    return _k(in1, tgt1)


def kernel(input, target, class_num):
    B, C, H, W = input.shape
    HW = H * W
    in1 = input.reshape(-1)
    tgt1 = target.reshape(-1)
    partials = _iou_counts(in1, tgt1, B, C, HW)
    p = partials.reshape(-1, 3, _LANES, _CPAD).sum(axis=(0, 2))
    intersect = p[0, :C]
    union = p[1, :C] + p[2, :C] - intersect
    eps = 1e-4
    return (intersect + eps) / (union + eps)
